# Initial kernel scaffold; baseline (speedup 1.0000x reference)
import jax
import jax.numpy as jnp
from jax.experimental import pallas as pl


def kernel(subed, sub):
    raise NotImplementedError("write your pallas kernel here")


# speedup vs baseline: 21.0990x; 21.0990x over previous
import functools

import jax
import jax.numpy as jnp
from jax import lax
from jax.experimental import pallas as pl
from jax.experimental.pallas import tpu as pltpu
from jax.experimental.pallas import tpu_sc as plsc

W = 4096
F = 64
D = 8
K = 2 * D + 1
B = 8
WP = W + 2 * D

NUM_WORKERS = 32
ROWS_PER_WORKER = (B * W) // NUM_WORKERS
WORKERS_PER_BATCH = W // ROWS_PER_WORKER
CH = 128
NCHUNK = ROWS_PER_WORKER // CH

DELAYS = [0] + [i for i in range(1, D + 1)] + [-i for i in range(1, D + 1)]

LANES = 16
FGROUPS = F // LANES


def _sc_body(subed_hbm, subpad_hbm, out_hbm, sub_buf, subed_buf, out_buf):
    wid = lax.axis_index("s") * 2 + lax.axis_index("c")
    b = wid // WORKERS_PER_BATCH
    q = wid % WORKERS_PER_BATCH
    w0 = q * ROWS_PER_WORKER
    row0 = b * W + w0
    pad0 = b * WP + w0

    def chunk_body(c, _):
        src0 = pad0 + c * CH
        dst0 = row0 + c * CH
        pltpu.sync_copy(subpad_hbm.at[pl.ds(src0, CH + 2 * D)], sub_buf)
        pltpu.sync_copy(subed_hbm.at[pl.ds(dst0, CH)], subed_buf)

        def row_body(i, _):
            for f in range(FGROUPS):
                fs = pl.ds(f * LANES, LANES)
                x = subed_buf[i, fs]
                best = x - sub_buf[i + D, fs]
                besta = jnp.abs(best)
                for d in DELAYS[1:]:
                    r = x - sub_buf[i + D + d, fs]
                    ra = jnp.abs(r)
                    m = ra < besta
                    best = jnp.where(m, r, best)
                    besta = jnp.where(m, ra, besta)
                out_buf[i, fs] = best
            return 0

        lax.fori_loop(0, CH, row_body, 0)
        pltpu.sync_copy(out_buf, out_hbm.at[pl.ds(dst0, CH)])
        return 0

    lax.fori_loop(0, NCHUNK, chunk_body, 0)


@jax.jit
def kernel(subed, sub):
    sub_pad = jnp.pad(sub, ((0, 0), (D, D), (0, 0)), mode="edge")
    subed_flat = subed.reshape(B * W, F)
    subpad_flat = sub_pad.reshape(B * WP, F)

    mesh = plsc.VectorSubcoreMesh(core_axis_name="c", subcore_axis_name="s")
    out = pl.kernel(
        _sc_body,
        out_type=jax.ShapeDtypeStruct((B * W, F), jnp.float32),
        mesh=mesh,
        scratch_types=[
            pltpu.VMEM((CH + 2 * D, F), jnp.float32),
            pltpu.VMEM((CH, F), jnp.float32),
            pltpu.VMEM((CH, F), jnp.float32),
        ],
    )(subed_flat, subpad_flat)
    return out.reshape(B, W, F)
